# baseline (speedup 1.0000x reference)
import dataclasses
import functools

import jax
import jax.numpy as jnp
from jax import lax
from jax.experimental import pallas as pl
from jax.experimental.pallas import tpu as pltpu
from jax.experimental.pallas import tpu_sc as plsc

N_CORES = 2
N_SUBCORES = 16
N_WORKERS = N_CORES * N_SUBCORES
CH = 128
NPAD = 10240


def _sc_spmm(x, src, dst, vals, n_nodes):
    e_pad = src.shape[0]
    d = x.shape[1]
    blk = 10
    cpt_pair = e_pad // (N_SUBCORES * CH)
    cpt0 = 15 * cpt_pair // 16
    cpt1 = cpt_pair - cpt0
    assert e_pad == N_SUBCORES * CH * cpt_pair
    assert cpt0 % blk == 0 and cpt1 % blk == 0 and cpt0 % 2 == 0 and cpt1 % 2 == 0
    rows_per_tile = n_nodes // N_SUBCORES
    assert n_nodes == N_SUBCORES * rows_per_tile
    mesh = plsc.VectorSubcoreMesh(core_axis_name="c", subcore_axis_name="s")
    cp = pltpu.CompilerParams()
    if "needs_layout_passes" in pltpu.CompilerParams.__dataclass_fields__:
        cp = dataclasses.replace(cp, needs_layout_passes=False)
    if "use_tc_tiling_on_sc" in pltpu.CompilerParams.__dataclass_fields__:
        cp = dataclasses.replace(cp, use_tc_tiling_on_sc=False)

    @functools.partial(
        pl.kernel,
        compiler_params=cp,
        out_type=jax.ShapeDtypeStruct((N_CORES, n_nodes, d), jnp.float32),
        mesh=mesh,
        name=f"sc_spmm_d{d}",
        scratch_types=[
            pltpu.VMEM((blk * CH,), jnp.int32),
            pltpu.VMEM((blk * CH,), jnp.int32),
            pltpu.VMEM((blk * CH,), jnp.float32),
            pltpu.VMEM((CH,), jnp.int32),
            pltpu.VMEM((CH,), jnp.int32),
            pltpu.VMEM((CH, d), jnp.float32),
            pltpu.VMEM((CH, d), jnp.float32),
            pltpu.VMEM_SHARED((n_nodes, d), jnp.float32),
            pltpu.SemaphoreType.DMA,
            pltpu.SemaphoreType.DMA,
            pltpu.SemaphoreType.DMA,
            pltpu.SemaphoreType.DMA,
        ],
    )
    def spmm_kernel(x_hbm, src_hbm, dst_hbm, vals_hbm, zeros_hbm, out_hbm,
                    srcv, dstv, valsv, dchunk0, dchunk1, rows0, rows1, acc,
                    gsem0, gsem1, ssem0, ssem1):
        c = lax.axis_index("c")
        s = lax.axis_index("s")
        pltpu.sync_copy(zeros_hbm.at[pl.ds(s * rows_per_tile, rows_per_tile)],
                        acc.at[pl.ds(s * rows_per_tile, rows_per_tile)])
        plsc.subcore_barrier()
        cpt = jnp.where(c == 0, cpt0, cpt1)
        tile_chunk0 = jnp.where(c == 0, s * cpt0,
                                N_SUBCORES * cpt0 + s * cpt1)

        def refill(k):
            e0 = (tile_chunk0 + k * blk) * CH
            pltpu.sync_copy(src_hbm.at[pl.ds(e0, blk * CH)], srcv)
            pltpu.sync_copy(dst_hbm.at[pl.ds(e0, blk * CH)], dstv)
            pltpu.sync_copy(vals_hbm.at[pl.ds(e0, blk * CH)], valsv)

        dchunk = (dchunk0, dchunk1)
        rows = (rows0, rows1)
        gsem = (gsem0, gsem1)
        ssem = (ssem0, ssem1)

        def gather_start(g, b):
            lg = lax.rem(g, blk) * CH
            pltpu.async_copy(x_hbm.at[srcv.at[pl.ds(lg, CH)]],
                             rows[b], gsem[b])

        def gather_wait(b):
            pltpu.make_async_copy(x_hbm.at[srcv.at[pl.ds(0, CH)]],
                                  rows[b], gsem[b]).wait()

        def scatter_start(g, b):
            lg = lax.rem(g, blk) * CH
            for t in range(CH // 16):
                dchunk[b][pl.ds(t * 16, 16)] = dstv[pl.ds(lg + t * 16, 16)]
            pltpu.async_copy(rows[b], acc.at[dchunk[b]], ssem[b], add=True)

        def scatter_wait(b):
            pltpu.make_async_copy(rows[b], acc.at[dchunk[b]], ssem[b]).wait()

        def scale(g, b):
            lg = lax.rem(g, blk) * CH

            @pl.loop(0, CH)
            def _edge(e):
                vv = plsc.load_gather(valsv,
                                      [jnp.full((16,), lg + e, jnp.int32)])
                r = rows[b]
                for t in range(d // 16):
                    sl = pl.ds(t * 16, 16)
                    r[e, sl] = r[e, sl] * vv

        refill(jnp.int32(0))
        gather_start(0, 0)
        gather_start(1, 1)

        @pl.loop(0, cpt, step=2)
        def _pair(g):
            gather_wait(0)
            scale(g, 0)
            scatter_start(g, 0)
            gather_wait(1)
            scale(g + 1, 1)
            scatter_start(g + 1, 1)
            scatter_wait(0)
            scatter_wait(1)

            @pl.when(g + 2 < cpt)
            def _():
                @pl.when(lax.rem(g + 2, blk) == 0)
                def _():
                    refill((g + 2) // blk)

                gather_start(g + 2, 0)
                gather_start(g + 3, 1)

        plsc.subcore_barrier()
        pltpu.sync_copy(acc.at[pl.ds(s * rows_per_tile, rows_per_tile)],
                        out_hbm.at[c, pl.ds(s * rows_per_tile, rows_per_tile)])

    zeros = jnp.zeros((n_nodes, d), jnp.float32)
    return spmm_kernel(x, src, dst, vals, zeros)


def _tc_matmul(x, w):
    n, k = x.shape
    m = w.shape[1]
    blk = 1024

    def body(x_ref, w_ref, o_ref):
        o_ref[...] = lax.dot_general(
            x_ref[...], w_ref[...], (((1,), (0,)), ((), ())),
            precision=lax.Precision.HIGHEST,
            preferred_element_type=jnp.float32)

    return pl.pallas_call(
        body,
        grid=(n // blk,),
        in_specs=[pl.BlockSpec((blk, k), lambda i: (i, 0)),
                  pl.BlockSpec((k, m), lambda i: (0, 0))],
        out_specs=pl.BlockSpec((blk, m), lambda i: (i, 0)),
        out_shape=jax.ShapeDtypeStruct((n, m), jnp.float32),
    )(x, w)


def _tc_add_relu_matmul(p, w):
    _, n, k = p.shape
    m = w.shape[1]
    blk = 1024

    def body(p_ref, w_ref, o_ref):
        h = jnp.maximum(p_ref[0] + p_ref[1], 0.0)
        o_ref[...] = lax.dot_general(
            h, w_ref[...], (((1,), (0,)), ((), ())),
            precision=lax.Precision.HIGHEST,
            preferred_element_type=jnp.float32)

    return pl.pallas_call(
        body,
        grid=(n // blk,),
        in_specs=[pl.BlockSpec((2, blk, k), lambda i: (0, i, 0)),
                  pl.BlockSpec((k, m), lambda i: (0, 0))],
        out_specs=pl.BlockSpec((blk, m), lambda i: (i, 0)),
        out_shape=jax.ShapeDtypeStruct((n, m), jnp.float32),
    )(p, w)


def _tc_add_softmax(q):
    _, n, m = q.shape
    blk = 1024

    def body(q_ref, o_ref):
        logits = q_ref[0] + q_ref[1]
        z = logits - jnp.max(logits, axis=1, keepdims=True)
        e = jnp.exp(z)
        o_ref[...] = e / jnp.sum(e, axis=1, keepdims=True)

    return pl.pallas_call(
        body,
        grid=(n // blk,),
        in_specs=[pl.BlockSpec((2, blk, m), lambda i: (0, i, 0))],
        out_specs=pl.BlockSpec((blk, m), lambda i: (i, 0)),
        out_shape=jax.ShapeDtypeStruct((n, m), jnp.float32),
    )(q)


def kernel(embeddings, edge_index, adj_vals, W1, W2):
    n = embeddings.shape[0]
    e = adj_vals.shape[0]
    src = edge_index[0].astype(jnp.int32)
    dst = edge_index[1].astype(jnp.int32)
    vals = adj_vals.astype(jnp.float32)
    unit = N_WORKERS * CH * 2
    e_pad = ((e + unit - 1) // unit) * unit
    if e_pad != e:
        pad = e_pad - e
        src = jnp.concatenate([src, jnp.zeros((pad,), jnp.int32)])
        dst = jnp.concatenate([dst, jnp.zeros((pad,), jnp.int32)])
        vals = jnp.concatenate([vals, jnp.zeros((pad,), jnp.float32)])

    x_pad = jnp.pad(embeddings, ((0, NPAD - n), (0, 0)))
    y1 = _tc_matmul(x_pad, W1)
    p = _sc_spmm(y1, src, dst, vals, NPAD)
    y2 = _tc_add_relu_matmul(p, W2)
    q = _sc_spmm(y2, src, dst, vals, NPAD)
    return _tc_add_softmax(q)[:n]


# speedup vs baseline: 1.0502x; 1.0502x over previous
import dataclasses
import functools

import jax
import jax.numpy as jnp
from jax import lax
from jax.experimental import pallas as pl
from jax.experimental.pallas import tpu as pltpu
from jax.experimental.pallas import tpu_sc as plsc

N_CORES = 2
N_SUBCORES = 16
N_WORKERS = N_CORES * N_SUBCORES
CH = 128
NPAD = 10240


def _sc_spmm(x, src, dst, vals, n_nodes):
    e_pad = src.shape[0]
    d = x.shape[1]
    blk = 8
    cpt_pair = e_pad // (N_SUBCORES * CH)
    cpt0 = 9 * cpt_pair // 10
    cpt1 = cpt_pair - cpt0
    assert e_pad == N_SUBCORES * CH * cpt_pair
    assert cpt0 % blk == 0 and cpt1 % blk == 0 and cpt0 % 2 == 0 and cpt1 % 2 == 0
    rows_per_tile = n_nodes // N_SUBCORES
    assert n_nodes == N_SUBCORES * rows_per_tile
    mesh = plsc.VectorSubcoreMesh(core_axis_name="c", subcore_axis_name="s")
    cp = pltpu.CompilerParams()
    if "needs_layout_passes" in pltpu.CompilerParams.__dataclass_fields__:
        cp = dataclasses.replace(cp, needs_layout_passes=False)
    if "use_tc_tiling_on_sc" in pltpu.CompilerParams.__dataclass_fields__:
        cp = dataclasses.replace(cp, use_tc_tiling_on_sc=False)

    @functools.partial(
        pl.kernel,
        compiler_params=cp,
        out_type=jax.ShapeDtypeStruct((N_CORES, n_nodes, d), jnp.float32),
        mesh=mesh,
        name=f"sc_spmm_d{d}",
        scratch_types=[
            pltpu.VMEM((blk * CH,), jnp.int32),
            pltpu.VMEM((blk * CH,), jnp.int32),
            pltpu.VMEM((blk * CH,), jnp.float32),
            pltpu.VMEM((CH,), jnp.int32),
            pltpu.VMEM((CH,), jnp.int32),
            pltpu.VMEM((CH, d), jnp.float32),
            pltpu.VMEM((CH, d), jnp.float32),
            pltpu.VMEM_SHARED((n_nodes, d), jnp.float32),
            pltpu.SemaphoreType.DMA,
            pltpu.SemaphoreType.DMA,
            pltpu.SemaphoreType.DMA,
            pltpu.SemaphoreType.DMA,
        ],
    )
    def spmm_kernel(x_hbm, src_hbm, dst_hbm, vals_hbm, out_hbm,
                    srcv, dstv, valsv, dchunk0, dchunk1, rows0, rows1, acc,
                    gsem0, gsem1, ssem0, ssem1):
        c = lax.axis_index("c")
        s = lax.axis_index("s")
        zv = jnp.zeros((16,), jnp.float32)

        @pl.loop(0, CH)
        def _zrow(r):
            for t in range(d // 16):
                rows0[r, pl.ds(t * 16, 16)] = zv

        @pl.loop(0, rows_per_tile // CH)
        def _zacc(i):
            pltpu.sync_copy(
                rows0, acc.at[pl.ds(s * rows_per_tile + i * CH, CH)])

        plsc.subcore_barrier()
        cpt = jnp.where(c == 0, cpt0, cpt1)
        tile_chunk0 = jnp.where(c == 0, s * cpt0,
                                N_SUBCORES * cpt0 + s * cpt1)

        def refill(k):
            e0 = (tile_chunk0 + k * blk) * CH
            pltpu.sync_copy(src_hbm.at[pl.ds(e0, blk * CH)], srcv)
            pltpu.sync_copy(dst_hbm.at[pl.ds(e0, blk * CH)], dstv)
            pltpu.sync_copy(vals_hbm.at[pl.ds(e0, blk * CH)], valsv)

        dchunk = (dchunk0, dchunk1)
        rows = (rows0, rows1)
        gsem = (gsem0, gsem1)
        ssem = (ssem0, ssem1)

        def gather_start(g, b):
            lg = lax.rem(g, blk) * CH
            pltpu.async_copy(x_hbm.at[srcv.at[pl.ds(lg, CH)]],
                             rows[b], gsem[b])

        def gather_wait(b):
            pltpu.make_async_copy(x_hbm.at[srcv.at[pl.ds(0, CH)]],
                                  rows[b], gsem[b]).wait()

        def scatter_start(g, b):
            lg = lax.rem(g, blk) * CH
            for t in range(CH // 16):
                dchunk[b][pl.ds(t * 16, 16)] = dstv[pl.ds(lg + t * 16, 16)]
            pltpu.async_copy(rows[b], acc.at[dchunk[b]], ssem[b], add=True)

        def scatter_wait(b):
            pltpu.make_async_copy(rows[b], acc.at[dchunk[b]], ssem[b]).wait()

        def scale(g, b):
            lg = lax.rem(g, blk) * CH

            @pl.loop(0, CH)
            def _edge(e):
                vv = plsc.load_gather(valsv,
                                      [jnp.full((16,), lg + e, jnp.int32)])
                r = rows[b]
                for t in range(d // 16):
                    sl = pl.ds(t * 16, 16)
                    r[e, sl] = r[e, sl] * vv

        refill(jnp.int32(0))
        gather_start(0, 0)
        gather_start(1, 1)

        @pl.loop(0, cpt, step=2)
        def _pair(g):
            gather_wait(0)
            scale(g, 0)
            scatter_start(g, 0)
            gather_wait(1)
            scale(g + 1, 1)
            scatter_start(g + 1, 1)
            scatter_wait(0)
            scatter_wait(1)

            @pl.when(g + 2 < cpt)
            def _():
                @pl.when(lax.rem(g + 2, blk) == 0)
                def _():
                    refill((g + 2) // blk)

                gather_start(g + 2, 0)
                gather_start(g + 3, 1)

        plsc.subcore_barrier()
        pltpu.sync_copy(acc.at[pl.ds(s * rows_per_tile, rows_per_tile)],
                        out_hbm.at[c, pl.ds(s * rows_per_tile, rows_per_tile)])

    return spmm_kernel(x, src, dst, vals)


def _tc_matmul(x, w):
    n, k = x.shape
    m = w.shape[1]
    blk = 1024

    def body(x_ref, w_ref, o_ref):
        o_ref[...] = lax.dot_general(
            x_ref[...], w_ref[...], (((1,), (0,)), ((), ())),
            precision=lax.Precision.HIGHEST,
            preferred_element_type=jnp.float32)

    return pl.pallas_call(
        body,
        grid=(n // blk,),
        in_specs=[pl.BlockSpec((blk, k), lambda i: (i, 0)),
                  pl.BlockSpec((k, m), lambda i: (0, 0))],
        out_specs=pl.BlockSpec((blk, m), lambda i: (i, 0)),
        out_shape=jax.ShapeDtypeStruct((n, m), jnp.float32),
    )(x, w)


def _tc_add_relu_matmul(p, w):
    _, n, k = p.shape
    m = w.shape[1]
    blk = 1024

    def body(p_ref, w_ref, o_ref):
        h = jnp.maximum(p_ref[0] + p_ref[1], 0.0)
        o_ref[...] = lax.dot_general(
            h, w_ref[...], (((1,), (0,)), ((), ())),
            precision=lax.Precision.HIGHEST,
            preferred_element_type=jnp.float32)

    return pl.pallas_call(
        body,
        grid=(n // blk,),
        in_specs=[pl.BlockSpec((2, blk, k), lambda i: (0, i, 0)),
                  pl.BlockSpec((k, m), lambda i: (0, 0))],
        out_specs=pl.BlockSpec((blk, m), lambda i: (i, 0)),
        out_shape=jax.ShapeDtypeStruct((n, m), jnp.float32),
    )(p, w)


def _tc_add_softmax(q):
    _, n, m = q.shape
    blk = 1024

    def body(q_ref, o_ref):
        logits = q_ref[0] + q_ref[1]
        z = logits - jnp.max(logits, axis=1, keepdims=True)
        e = jnp.exp(z)
        o_ref[...] = e / jnp.sum(e, axis=1, keepdims=True)

    return pl.pallas_call(
        body,
        grid=(n // blk,),
        in_specs=[pl.BlockSpec((2, blk, m), lambda i: (0, i, 0))],
        out_specs=pl.BlockSpec((blk, m), lambda i: (i, 0)),
        out_shape=jax.ShapeDtypeStruct((n, m), jnp.float32),
    )(q)


def kernel(embeddings, edge_index, adj_vals, W1, W2):
    n = embeddings.shape[0]
    e = adj_vals.shape[0]
    src = edge_index[0].astype(jnp.int32)
    dst = edge_index[1].astype(jnp.int32)
    vals = adj_vals.astype(jnp.float32)
    unit = N_WORKERS * CH * 2
    e_pad = ((e + unit - 1) // unit) * unit
    if e_pad != e:
        pad = e_pad - e
        src = jnp.concatenate([src, jnp.zeros((pad,), jnp.int32)])
        dst = jnp.concatenate([dst, jnp.zeros((pad,), jnp.int32)])
        vals = jnp.concatenate([vals, jnp.zeros((pad,), jnp.float32)])

    x_pad = jnp.pad(embeddings, ((0, NPAD - n), (0, 0)))
    y1 = _tc_matmul(x_pad, W1)
    p = _sc_spmm(y1, src, dst, vals, NPAD)
    y2 = _tc_add_relu_matmul(p, W2)
    q = _sc_spmm(y2, src, dst, vals, NPAD)
    return _tc_add_softmax(q)[:n]


# speedup vs baseline: 1.0944x; 1.0421x over previous
import dataclasses
import functools

import jax
import jax.numpy as jnp
from jax import lax
from jax.experimental import pallas as pl
from jax.experimental.pallas import tpu as pltpu
from jax.experimental.pallas import tpu_sc as plsc

N_CORES = 2
N_SUBCORES = 16
N_WORKERS = N_CORES * N_SUBCORES
CH = 128
NPAD = 10240


def _sc_spmm(x, src, dst, vals, n_nodes):
    e_pad = src.shape[0]
    d = x.shape[1]
    blk = 10
    cpt_pair = e_pad // (N_SUBCORES * CH)
    cpt0 = 7 * cpt_pair // 8
    cpt1 = cpt_pair - cpt0
    assert e_pad == N_SUBCORES * CH * cpt_pair
    assert cpt0 % blk == 0 and cpt1 % blk == 0 and cpt0 % 2 == 0 and cpt1 % 2 == 0
    rows_per_tile = n_nodes // N_SUBCORES
    assert n_nodes == N_SUBCORES * rows_per_tile
    mesh = plsc.VectorSubcoreMesh(core_axis_name="c", subcore_axis_name="s")
    cp = pltpu.CompilerParams()
    if "needs_layout_passes" in pltpu.CompilerParams.__dataclass_fields__:
        cp = dataclasses.replace(cp, needs_layout_passes=False)
    if "use_tc_tiling_on_sc" in pltpu.CompilerParams.__dataclass_fields__:
        cp = dataclasses.replace(cp, use_tc_tiling_on_sc=False)

    @functools.partial(
        pl.kernel,
        compiler_params=cp,
        out_type=jax.ShapeDtypeStruct((N_CORES, n_nodes, d), jnp.float32),
        mesh=mesh,
        name=f"sc_spmm_d{d}",
        scratch_types=[
            pltpu.VMEM((blk * CH,), jnp.int32),
            pltpu.VMEM((blk * CH,), jnp.int32),
            pltpu.VMEM((blk * CH,), jnp.float32),
            pltpu.VMEM((CH,), jnp.int32),
            pltpu.VMEM((CH,), jnp.int32),
            pltpu.VMEM((CH, d), jnp.float32),
            pltpu.VMEM((CH, d), jnp.float32),
            pltpu.VMEM_SHARED((n_nodes, d), jnp.float32),
            pltpu.SemaphoreType.DMA,
            pltpu.SemaphoreType.DMA,
            pltpu.SemaphoreType.DMA,
            pltpu.SemaphoreType.DMA,
        ],
    )
    def spmm_kernel(x_hbm, src_hbm, dst_hbm, vals_hbm, out_hbm,
                    srcv, dstv, valsv, dchunk0, dchunk1, rows0, rows1, acc,
                    gsem0, gsem1, ssem0, ssem1):
        c = lax.axis_index("c")
        s = lax.axis_index("s")
        zv = jnp.zeros((16,), jnp.float32)

        @pl.loop(0, CH)
        def _zrow(r):
            for t in range(d // 16):
                rows0[r, pl.ds(t * 16, 16)] = zv

        @pl.loop(0, rows_per_tile // CH)
        def _zacc(i):
            pltpu.sync_copy(
                rows0, acc.at[pl.ds(s * rows_per_tile + i * CH, CH)])

        plsc.subcore_barrier()
        cpt = jnp.where(c == 0, cpt0, cpt1)
        tile_chunk0 = jnp.where(c == 0, s * cpt0,
                                N_SUBCORES * cpt0 + s * cpt1)

        def refill(k):
            e0 = (tile_chunk0 + k * blk) * CH
            pltpu.sync_copy(src_hbm.at[pl.ds(e0, blk * CH)], srcv)
            pltpu.sync_copy(dst_hbm.at[pl.ds(e0, blk * CH)], dstv)
            pltpu.sync_copy(vals_hbm.at[pl.ds(e0, blk * CH)], valsv)

        dchunk = (dchunk0, dchunk1)
        rows = (rows0, rows1)
        gsem = (gsem0, gsem1)
        ssem = (ssem0, ssem1)

        def gather_start(g, b):
            lg = lax.rem(g, blk) * CH
            pltpu.async_copy(x_hbm.at[srcv.at[pl.ds(lg, CH)]],
                             rows[b], gsem[b])

        def gather_wait(b):
            pltpu.make_async_copy(x_hbm.at[srcv.at[pl.ds(0, CH)]],
                                  rows[b], gsem[b]).wait()

        def scatter_start(g, b):
            lg = lax.rem(g, blk) * CH
            for t in range(CH // 16):
                dchunk[b][pl.ds(t * 16, 16)] = dstv[pl.ds(lg + t * 16, 16)]
            pltpu.async_copy(rows[b], acc.at[dchunk[b]], ssem[b], add=True)

        def scatter_wait(b):
            pltpu.make_async_copy(rows[b], acc.at[dchunk[b]], ssem[b]).wait()

        def scale(g, b):
            lg = lax.rem(g, blk) * CH

            @pl.loop(0, CH)
            def _edge(e):
                vv = plsc.load_gather(valsv,
                                      [jnp.full((16,), lg + e, jnp.int32)])
                r = rows[b]
                for t in range(d // 16):
                    sl = pl.ds(t * 16, 16)
                    r[e, sl] = r[e, sl] * vv

        refill(jnp.int32(0))
        gather_start(0, 0)
        gather_start(1, 1)

        @pl.loop(0, cpt, step=2)
        def _pair(g):
            gather_wait(0)
            scale(g, 0)
            scatter_start(g, 0)
            gather_wait(1)
            scale(g + 1, 1)
            scatter_start(g + 1, 1)
            scatter_wait(0)
            scatter_wait(1)

            @pl.when(g + 2 < cpt)
            def _():
                @pl.when(lax.rem(g + 2, blk) == 0)
                def _():
                    refill((g + 2) // blk)

                gather_start(g + 2, 0)
                gather_start(g + 3, 1)

        plsc.subcore_barrier()
        pltpu.sync_copy(acc.at[pl.ds(s * rows_per_tile, rows_per_tile)],
                        out_hbm.at[c, pl.ds(s * rows_per_tile, rows_per_tile)])

    return spmm_kernel(x, src, dst, vals)


def _tc_matmul(x, w):
    n, k = x.shape
    m = w.shape[1]
    blk = 1024

    def body(x_ref, w_ref, o_ref):
        o_ref[...] = lax.dot_general(
            x_ref[...], w_ref[...], (((1,), (0,)), ((), ())),
            precision=lax.Precision.HIGHEST,
            preferred_element_type=jnp.float32)

    return pl.pallas_call(
        body,
        grid=(n // blk,),
        in_specs=[pl.BlockSpec((blk, k), lambda i: (i, 0)),
                  pl.BlockSpec((k, m), lambda i: (0, 0))],
        out_specs=pl.BlockSpec((blk, m), lambda i: (i, 0)),
        out_shape=jax.ShapeDtypeStruct((n, m), jnp.float32),
    )(x, w)


def _tc_add_relu_matmul(p, w):
    _, n, k = p.shape
    m = w.shape[1]
    blk = 1024

    def body(p_ref, w_ref, o_ref):
        h = jnp.maximum(p_ref[0] + p_ref[1], 0.0)
        o_ref[...] = lax.dot_general(
            h, w_ref[...], (((1,), (0,)), ((), ())),
            precision=lax.Precision.HIGHEST,
            preferred_element_type=jnp.float32)

    return pl.pallas_call(
        body,
        grid=(n // blk,),
        in_specs=[pl.BlockSpec((2, blk, k), lambda i: (0, i, 0)),
                  pl.BlockSpec((k, m), lambda i: (0, 0))],
        out_specs=pl.BlockSpec((blk, m), lambda i: (i, 0)),
        out_shape=jax.ShapeDtypeStruct((n, m), jnp.float32),
    )(p, w)


def _tc_add_softmax(q):
    _, n, m = q.shape
    blk = 1024

    def body(q_ref, o_ref):
        logits = q_ref[0] + q_ref[1]
        z = logits - jnp.max(logits, axis=1, keepdims=True)
        e = jnp.exp(z)
        o_ref[...] = e / jnp.sum(e, axis=1, keepdims=True)

    return pl.pallas_call(
        body,
        grid=(n // blk,),
        in_specs=[pl.BlockSpec((2, blk, m), lambda i: (0, i, 0))],
        out_specs=pl.BlockSpec((blk, m), lambda i: (i, 0)),
        out_shape=jax.ShapeDtypeStruct((n, m), jnp.float32),
    )(q)


def kernel(embeddings, edge_index, adj_vals, W1, W2):
    n = embeddings.shape[0]
    e = adj_vals.shape[0]
    src = edge_index[0].astype(jnp.int32)
    dst = edge_index[1].astype(jnp.int32)
    vals = adj_vals.astype(jnp.float32)
    unit = N_WORKERS * CH * 2
    e_pad = ((e + unit - 1) // unit) * unit
    if e_pad != e:
        pad = e_pad - e
        src = jnp.concatenate([src, jnp.zeros((pad,), jnp.int32)])
        dst = jnp.concatenate([dst, jnp.zeros((pad,), jnp.int32)])
        vals = jnp.concatenate([vals, jnp.zeros((pad,), jnp.float32)])

    x_pad = jnp.pad(embeddings, ((0, NPAD - n), (0, 0)))
    y1 = _tc_matmul(x_pad, W1)
    p = _sc_spmm(y1, src, dst, vals, NPAD)
    y2 = _tc_add_relu_matmul(p, W2)
    q = _sc_spmm(y2, src, dst, vals, NPAD)
    return _tc_add_softmax(q)[:n]


# speedup vs baseline: 1.1285x; 1.0311x over previous
import dataclasses
import functools

import jax
import jax.numpy as jnp
from jax import lax
from jax.experimental import pallas as pl
from jax.experimental.pallas import tpu as pltpu
from jax.experimental.pallas import tpu_sc as plsc

N_CORES = 2
N_SUBCORES = 16
N_WORKERS = N_CORES * N_SUBCORES
CH = 128
NPAD = 10240


def _sc_spmm(x, src, dst, vals, n_nodes, split16):
    e_pad = src.shape[0]
    d = x.shape[1]
    blk = 10
    cpt_pair = e_pad // (N_SUBCORES * CH)
    cpt0 = split16 * cpt_pair // 16
    cpt1 = cpt_pair - cpt0
    assert e_pad == N_SUBCORES * CH * cpt_pair
    assert cpt0 % blk == 0 and cpt1 % blk == 0 and cpt0 % 2 == 0 and cpt1 % 2 == 0
    rows_per_tile = n_nodes // N_SUBCORES
    assert n_nodes == N_SUBCORES * rows_per_tile
    mesh = plsc.VectorSubcoreMesh(core_axis_name="c", subcore_axis_name="s")
    cp = pltpu.CompilerParams()
    if "needs_layout_passes" in pltpu.CompilerParams.__dataclass_fields__:
        cp = dataclasses.replace(cp, needs_layout_passes=False)
    if "use_tc_tiling_on_sc" in pltpu.CompilerParams.__dataclass_fields__:
        cp = dataclasses.replace(cp, use_tc_tiling_on_sc=False)

    @functools.partial(
        pl.kernel,
        compiler_params=cp,
        out_type=jax.ShapeDtypeStruct((N_CORES, n_nodes, d), jnp.float32),
        mesh=mesh,
        name=f"sc_spmm_d{d}",
        scratch_types=[
            pltpu.VMEM((blk * CH,), jnp.int32),
            pltpu.VMEM((blk * CH,), jnp.int32),
            pltpu.VMEM((blk * CH,), jnp.float32),
            pltpu.VMEM((CH,), jnp.int32),
            pltpu.VMEM((CH,), jnp.int32),
            pltpu.VMEM((CH, d), jnp.float32),
            pltpu.VMEM((CH, d), jnp.float32),
            pltpu.VMEM_SHARED((n_nodes, d), jnp.float32),
            pltpu.SemaphoreType.DMA,
            pltpu.SemaphoreType.DMA,
            pltpu.SemaphoreType.DMA,
            pltpu.SemaphoreType.DMA,
        ],
    )
    def spmm_kernel(x_hbm, src_hbm, dst_hbm, vals_hbm, out_hbm,
                    srcv, dstv, valsv, dchunk0, dchunk1, rows0, rows1, acc,
                    gsem0, gsem1, ssem0, ssem1):
        c = lax.axis_index("c")
        s = lax.axis_index("s")
        zv = jnp.zeros((16,), jnp.float32)

        @pl.loop(0, CH)
        def _zrow(r):
            for t in range(d // 16):
                rows0[r, pl.ds(t * 16, 16)] = zv

        @pl.loop(0, rows_per_tile // CH)
        def _zacc(i):
            pltpu.sync_copy(
                rows0, acc.at[pl.ds(s * rows_per_tile + i * CH, CH)])

        plsc.subcore_barrier()
        cpt = jnp.where(c == 0, cpt0, cpt1)
        tile_chunk0 = jnp.where(c == 0, s * cpt0,
                                N_SUBCORES * cpt0 + s * cpt1)

        def refill(k):
            e0 = (tile_chunk0 + k * blk) * CH
            pltpu.sync_copy(src_hbm.at[pl.ds(e0, blk * CH)], srcv)
            pltpu.sync_copy(dst_hbm.at[pl.ds(e0, blk * CH)], dstv)
            pltpu.sync_copy(vals_hbm.at[pl.ds(e0, blk * CH)], valsv)

        dchunk = (dchunk0, dchunk1)
        rows = (rows0, rows1)
        gsem = (gsem0, gsem1)
        ssem = (ssem0, ssem1)

        def gather_start(g, b):
            lg = lax.rem(g, blk) * CH
            pltpu.async_copy(x_hbm.at[srcv.at[pl.ds(lg, CH)]],
                             rows[b], gsem[b])

        def gather_wait(b):
            pltpu.make_async_copy(x_hbm.at[srcv.at[pl.ds(0, CH)]],
                                  rows[b], gsem[b]).wait()

        def scatter_start(g, b):
            lg = lax.rem(g, blk) * CH
            for t in range(CH // 16):
                dchunk[b][pl.ds(t * 16, 16)] = dstv[pl.ds(lg + t * 16, 16)]
            pltpu.async_copy(rows[b], acc.at[dchunk[b]], ssem[b], add=True)

        def scatter_wait(b):
            pltpu.make_async_copy(rows[b], acc.at[dchunk[b]], ssem[b]).wait()

        def scale(g, b):
            lg = lax.rem(g, blk) * CH

            @pl.loop(0, CH)
            def _edge(e):
                vv = plsc.load_gather(valsv,
                                      [jnp.full((16,), lg + e, jnp.int32)])
                r = rows[b]
                for t in range(d // 16):
                    sl = pl.ds(t * 16, 16)
                    r[e, sl] = r[e, sl] * vv

        refill(jnp.int32(0))
        gather_start(0, 0)
        gather_start(1, 1)

        @pl.loop(0, cpt, step=2)
        def _pair(g):
            gather_wait(0)
            scale(g, 0)
            scatter_start(g, 0)
            gather_wait(1)
            scale(g + 1, 1)
            scatter_start(g + 1, 1)
            scatter_wait(0)
            scatter_wait(1)

            @pl.when(g + 2 < cpt)
            def _():
                @pl.when(lax.rem(g + 2, blk) == 0)
                def _():
                    refill((g + 2) // blk)

                gather_start(g + 2, 0)
                gather_start(g + 3, 1)

        plsc.subcore_barrier()
        pltpu.sync_copy(acc.at[pl.ds(s * rows_per_tile, rows_per_tile)],
                        out_hbm.at[c, pl.ds(s * rows_per_tile, rows_per_tile)])

    return spmm_kernel(x, src, dst, vals)


def _tc_matmul(x, w):
    n, k = x.shape
    m = w.shape[1]
    blk = 1024

    def body(x_ref, w_ref, o_ref):
        o_ref[...] = lax.dot_general(
            x_ref[...], w_ref[...], (((1,), (0,)), ((), ())),
            precision=lax.Precision.HIGHEST,
            preferred_element_type=jnp.float32)

    return pl.pallas_call(
        body,
        grid=(n // blk,),
        in_specs=[pl.BlockSpec((blk, k), lambda i: (i, 0)),
                  pl.BlockSpec((k, m), lambda i: (0, 0))],
        out_specs=pl.BlockSpec((blk, m), lambda i: (i, 0)),
        out_shape=jax.ShapeDtypeStruct((n, m), jnp.float32),
    )(x, w)


def _tc_add_relu_matmul(p, w):
    _, n, k = p.shape
    m = w.shape[1]
    blk = 1024

    def body(p_ref, w_ref, o_ref):
        h = jnp.maximum(p_ref[0] + p_ref[1], 0.0)
        o_ref[...] = lax.dot_general(
            h, w_ref[...], (((1,), (0,)), ((), ())),
            precision=lax.Precision.HIGHEST,
            preferred_element_type=jnp.float32)

    return pl.pallas_call(
        body,
        grid=(n // blk,),
        in_specs=[pl.BlockSpec((2, blk, k), lambda i: (0, i, 0)),
                  pl.BlockSpec((k, m), lambda i: (0, 0))],
        out_specs=pl.BlockSpec((blk, m), lambda i: (i, 0)),
        out_shape=jax.ShapeDtypeStruct((n, m), jnp.float32),
    )(p, w)


def _tc_add_softmax(q):
    _, n, m = q.shape
    blk = 1024

    def body(q_ref, o_ref):
        logits = q_ref[0] + q_ref[1]
        z = logits - jnp.max(logits, axis=1, keepdims=True)
        e = jnp.exp(z)
        o_ref[...] = e / jnp.sum(e, axis=1, keepdims=True)

    return pl.pallas_call(
        body,
        grid=(n // blk,),
        in_specs=[pl.BlockSpec((2, blk, m), lambda i: (0, i, 0))],
        out_specs=pl.BlockSpec((blk, m), lambda i: (i, 0)),
        out_shape=jax.ShapeDtypeStruct((n, m), jnp.float32),
    )(q)


def kernel(embeddings, edge_index, adj_vals, W1, W2):
    n = embeddings.shape[0]
    e = adj_vals.shape[0]
    src = edge_index[0].astype(jnp.int32)
    dst = edge_index[1].astype(jnp.int32)
    vals = adj_vals.astype(jnp.float32)
    unit = N_WORKERS * CH * 2
    e_pad = ((e + unit - 1) // unit) * unit
    if e_pad != e:
        pad = e_pad - e
        src = jnp.concatenate([src, jnp.zeros((pad,), jnp.int32)])
        dst = jnp.concatenate([dst, jnp.zeros((pad,), jnp.int32)])
        vals = jnp.concatenate([vals, jnp.zeros((pad,), jnp.float32)])

    x_pad = jnp.pad(embeddings, ((0, NPAD - n), (0, 0)))
    y1 = _tc_matmul(x_pad, W1)
    p = _sc_spmm(y1, src, dst, vals, NPAD, 14)
    y2 = _tc_add_relu_matmul(p, W2)
    q = _sc_spmm(y2, src, dst, vals, NPAD, 13)
    return _tc_add_softmax(q)[:n]
